# baseline (speedup 1.0000x reference)
import jax
import jax.numpy as jnp
from jax.experimental import pallas as pl
from jax.experimental.pallas import tpu as pltpu

_BM = 200


def _halfblock(adj_ref, b_ref, support_ref, out_ref):
    out = jnp.dot(adj_ref[...], support_ref[...], preferred_element_type=jnp.float32)
    out = out + b_ref[...]
    m = jnp.max(out, axis=1, keepdims=True)
    shifted = out - m
    lse = jnp.log(jnp.sum(jnp.exp(shifted), axis=1, keepdims=True))
    out_ref[...] = shifted - lse


def _gcn_kernel(x_ref, w_ref, b_ref, adj_a, adj_b, out_a, out_b, support_ref):
    @pl.when(pl.program_id(0) == 0)
    def _compute_support():
        support_ref[...] = jnp.dot(
            x_ref[...], w_ref[...], preferred_element_type=jnp.float32
        )

    _halfblock(adj_a, b_ref, support_ref, out_a)
    _halfblock(adj_b, b_ref, support_ref, out_b)


def kernel(x, adj, W, b):
    n, nfeat = x.shape
    nclass = W.shape[1]
    b2 = b.reshape(1, nclass)
    half = n // 2
    num_m = half // _BM
    off = num_m

    out_a, out_b = pl.pallas_call(
        _gcn_kernel,
        grid=(num_m,),
        in_specs=[
            pl.BlockSpec((n, nfeat), lambda i: (0, 0)),
            pl.BlockSpec((nfeat, nclass), lambda i: (0, 0)),
            pl.BlockSpec((1, nclass), lambda i: (0, 0)),
            pl.BlockSpec((_BM, n), lambda i: (i, 0)),
            pl.BlockSpec((_BM, n), lambda i: (i + off, 0)),
        ],
        out_specs=[
            pl.BlockSpec((_BM, nclass), lambda i: (i, 0)),
            pl.BlockSpec((_BM, nclass), lambda i: (i, 0)),
        ],
        out_shape=[
            jax.ShapeDtypeStruct((half, nclass), jnp.float32),
            jax.ShapeDtypeStruct((half, nclass), jnp.float32),
        ],
        scratch_shapes=[pltpu.VMEM((n, nclass), jnp.float32)],
        compiler_params=pltpu.CompilerParams(
            dimension_semantics=("arbitrary",),
        ),
    )(x, W, b2, adj, adj)

    return jnp.concatenate([out_a, out_b], axis=0)


# speedup vs baseline: 1.0358x; 1.0358x over previous
import jax
import jax.numpy as jnp
from jax import lax
from jax.experimental import pallas as pl
from jax.experimental.pallas import tpu as pltpu

_BM = 400


def _gcn_kernel(x_ref, w_ref, b_ref, adj_ref, out_ref, support_ref):
    @pl.when(pl.program_id(0) == 0)
    def _compute_support():
        support_ref[...] = jnp.dot(
            x_ref[...], w_ref[...], preferred_element_type=jnp.float32
        )

    out_t = lax.dot_general(
        support_ref[...],
        adj_ref[...],
        (((0,), (1,)), ((), ())),
        preferred_element_type=jnp.float32,
    )
    out_t = out_t + b_ref[...]
    m = jnp.max(out_t, axis=0, keepdims=True)
    shifted = out_t - m
    lse = jnp.log(jnp.sum(jnp.exp(shifted), axis=0, keepdims=True))
    out_ref[...] = (shifted - lse).T


def kernel(x, adj, W, b):
    n, nfeat = x.shape
    nclass = W.shape[1]
    b2 = b.reshape(nclass, 1)
    num_m = n // _BM

    return pl.pallas_call(
        _gcn_kernel,
        grid=(num_m,),
        in_specs=[
            pl.BlockSpec((n, nfeat), lambda i: (0, 0)),
            pl.BlockSpec((nfeat, nclass), lambda i: (0, 0)),
            pl.BlockSpec((nclass, 1), lambda i: (0, 0)),
            pl.BlockSpec((_BM, n), lambda i: (i, 0)),
        ],
        out_specs=pl.BlockSpec((_BM, nclass), lambda i: (i, 0)),
        out_shape=jax.ShapeDtypeStruct((n, nclass), jnp.float32),
        scratch_shapes=[pltpu.VMEM((n, nclass), jnp.float32)],
        compiler_params=pltpu.CompilerParams(
            dimension_semantics=("arbitrary",),
        ),
    )(x, W, b2, adj)


# speedup vs baseline: 1.0373x; 1.0014x over previous
import jax
import jax.numpy as jnp
from jax import lax
from jax.experimental import pallas as pl
from jax.experimental.pallas import tpu as pltpu

_BM = 400


def _gcn_kernel(x_ref, w_ref, b_ref, adj_ref, out_ref, support_ref):
    @pl.when(pl.program_id(0) == 0)
    def _compute_support():
        support_ref[...] = lax.dot_general(
            w_ref[...],
            x_ref[...],
            (((0,), (1,)), ((), ())),
            preferred_element_type=jnp.float32,
        )

    out_t = lax.dot_general(
        support_ref[...],
        adj_ref[...],
        (((1,), (1,)), ((), ())),
        preferred_element_type=jnp.float32,
    )
    out_t = out_t + b_ref[...]
    m = jnp.max(out_t, axis=0, keepdims=True)
    shifted = out_t - m
    lse = jnp.log(jnp.sum(jnp.exp(shifted), axis=0, keepdims=True))
    out_ref[...] = (shifted - lse).T


def kernel(x, adj, W, b):
    n, nfeat = x.shape
    nclass = W.shape[1]
    b2 = b.reshape(nclass, 1)
    num_m = n // _BM

    return pl.pallas_call(
        _gcn_kernel,
        grid=(num_m,),
        in_specs=[
            pl.BlockSpec((n, nfeat), lambda i: (0, 0)),
            pl.BlockSpec((nfeat, nclass), lambda i: (0, 0)),
            pl.BlockSpec((nclass, 1), lambda i: (0, 0)),
            pl.BlockSpec((_BM, n), lambda i: (i, 0)),
        ],
        out_specs=pl.BlockSpec((_BM, nclass), lambda i: (i, 0)),
        out_shape=jax.ShapeDtypeStruct((n, nclass), jnp.float32),
        scratch_shapes=[pltpu.VMEM((nclass, n), jnp.float32)],
        compiler_params=pltpu.CompilerParams(
            dimension_semantics=("arbitrary",),
        ),
    )(x, W, b2, adj)
